# Initial kernel scaffold; baseline (speedup 1.0000x reference)
import jax
import jax.numpy as jnp
from jax.experimental import pallas as pl


def kernel(x, token_table, pos_table, seg_table, ln_w, ln_b):
    raise NotImplementedError("write your pallas kernel here")


# speedup vs baseline: 8.2158x; 8.2158x over previous
import functools

import jax
import jax.numpy as jnp
from jax import lax
from jax.experimental import pallas as pl
from jax.experimental.pallas import tpu as pltpu
from jax.experimental.pallas import tpu_sc as plsc

_MAX_LEN = 150
_SEQ = _MAX_LEN * 2 + 2
_EMB = 64
_HALF = _MAX_LEN + 1

_GATHER_W = 128


def _sc_gather(token_table, flat_idx):
    n = flat_idx.shape[0]
    assert n % _GATHER_W == 0
    mesh = plsc.VectorSubcoreMesh(core_axis_name="c", subcore_axis_name="s")

    @functools.partial(
        pl.kernel,
        out_type=jax.ShapeDtypeStruct((n, _EMB), jnp.float32),
        mesh=mesh,
        compiler_params=pltpu.CompilerParams(use_tc_tiling_on_sc=False),
    )
    def gather_kernel(tab_hbm, idx_hbm, out_hbm):
        def body(i_vmem, o_vmem):
            pltpu.sync_copy(tab_hbm.at[i_vmem.at[0]], o_vmem)

        pltpu.emit_pipeline(
            body,
            grid=(n // _GATHER_W,),
            in_specs=[pl.BlockSpec((1, _GATHER_W), lambda i: (0, i))],
            out_specs=[pl.BlockSpec((_GATHER_W, _EMB), lambda i: (i, 0))],
            core_axis_name=("c", "s"),
            dimension_semantics=(pltpu.PARALLEL,),
        )(idx_hbm, out_hbm)

    return gather_kernel(token_table, flat_idx.reshape(1, n))


def _ln_body(tok_ref, pos_ref, seg_ref, w_ref, b_ref, out_ref):
    x = tok_ref[...]
    pos = pos_ref[...]
    seg = seg_ref[...]
    s_ids = lax.broadcasted_iota(jnp.int32, (_SEQ, 1), 0)
    seg_row = jnp.where(s_ids >= _HALF, seg[1:2, :], seg[0:1, :])
    emb = x + (pos + seg_row)[None, :, :]
    mean = jnp.mean(emb, axis=-1, keepdims=True)
    cent = emb - mean
    var = jnp.mean(cent * cent, axis=-1, keepdims=True)
    normed = cent * lax.rsqrt(var + 1e-5)
    out_ref[...] = normed * w_ref[...][None, :, :] + b_ref[...][None, :, :]


def _tc_add_ln(tok3, pos_table, seg_table, ln_w, ln_b):
    batch = tok3.shape[0]
    bblk = 32
    assert batch % bblk == 0
    return pl.pallas_call(
        _ln_body,
        out_shape=jax.ShapeDtypeStruct((batch, _SEQ, _EMB), jnp.float32),
        grid=(batch // bblk,),
        in_specs=[
            pl.BlockSpec((bblk, _SEQ, _EMB), lambda i: (i, 0, 0)),
            pl.BlockSpec((_SEQ, _EMB), lambda i: (0, 0)),
            pl.BlockSpec((2, _EMB), lambda i: (0, 0)),
            pl.BlockSpec((1, _EMB), lambda i: (0, 0)),
            pl.BlockSpec((1, _EMB), lambda i: (0, 0)),
        ],
        out_specs=pl.BlockSpec((bblk, _SEQ, _EMB), lambda i: (i, 0, 0)),
    )(tok3, pos_table, seg_table, ln_w.reshape(1, _EMB), ln_b.reshape(1, _EMB))


def kernel(x, token_table, pos_table, seg_table, ln_w, ln_b):
    batch = x.shape[0]
    flat_idx = x.reshape(-1)
    tok = _sc_gather(token_table, flat_idx)
    tok3 = tok.reshape(batch, _SEQ, _EMB)
    return _tc_add_ln(tok3, pos_table, seg_table, ln_w, ln_b)


# speedup vs baseline: 8.8670x; 1.0793x over previous
import functools

import jax
import jax.numpy as jnp
from jax import lax
from jax.experimental import pallas as pl
from jax.experimental.pallas import tpu as pltpu
from jax.experimental.pallas import tpu_sc as plsc

_MAX_LEN = 150
_SEQ = _MAX_LEN * 2 + 2
_EMB = 64
_HALF = _MAX_LEN + 1

_GATHER_W = 128


def _sc_gather(token_table, flat_idx):
    n = flat_idx.shape[0]
    assert n % _GATHER_W == 0
    mesh = plsc.VectorSubcoreMesh(core_axis_name="c", subcore_axis_name="s")

    @functools.partial(
        pl.kernel,
        out_type=jax.ShapeDtypeStruct((n, _EMB), jnp.float32),
        mesh=mesh,
        compiler_params=pltpu.CompilerParams(use_tc_tiling_on_sc=False),
    )
    def gather_kernel(tab_hbm, idx_hbm, out_hbm):
        def body(i_vmem, o_vmem):
            pltpu.sync_copy(tab_hbm.at[i_vmem.at[0]], o_vmem)

        pltpu.emit_pipeline(
            body,
            grid=(n // _GATHER_W,),
            in_specs=[pl.BlockSpec((1, _GATHER_W), lambda i: (0, i))],
            out_specs=[pl.BlockSpec((_GATHER_W, _EMB), lambda i: (i, 0))],
            core_axis_name=("c", "s"),
            dimension_semantics=(pltpu.PARALLEL,),
        )(idx_hbm, out_hbm)

    return gather_kernel(token_table, flat_idx.reshape(1, n))


_HSEQ = _SEQ // 2
_PK = 2 * _EMB
_PBATCH = 8


def _ln_body(tok_ref, comb_ref, w2_ref, b2_ref, out_ref):
    k_io = lax.broadcasted_iota(jnp.int32, (_PK, _PK), 0)
    l_io = lax.broadcasted_iota(jnp.int32, (_PK, _PK), 1)
    avg = jnp.where((k_io >= _EMB) == (l_io >= _EMB), 1.0 / _EMB, 0.0)
    avg = avg.astype(jnp.bfloat16)

    emb = tok_ref[...] + comb_ref[...]
    mean = jnp.dot(emb.astype(jnp.bfloat16), avg,
                   preferred_element_type=jnp.float32)
    sq = emb * emb
    meansq = jnp.dot(sq.astype(jnp.bfloat16), avg,
                     preferred_element_type=jnp.float32)
    var = meansq - mean * mean
    normed = (emb - mean) * lax.rsqrt(var + 1e-5)
    out_ref[...] = normed * w2_ref[...] + b2_ref[...]


def _tc_add_ln(tok2, comb, w2, b2):
    rows = tok2.shape[0]
    rblk = _PBATCH * _HSEQ
    assert rows % rblk == 0
    return pl.pallas_call(
        _ln_body,
        out_shape=jax.ShapeDtypeStruct((rows, _PK), jnp.float32),
        grid=(rows // rblk,),
        in_specs=[
            pl.BlockSpec((rblk, _PK), lambda i: (i, 0)),
            pl.BlockSpec((rblk, _PK), lambda i: (0, 0)),
            pl.BlockSpec((1, _PK), lambda i: (0, 0)),
            pl.BlockSpec((1, _PK), lambda i: (0, 0)),
        ],
        out_specs=pl.BlockSpec((rblk, _PK), lambda i: (i, 0)),
    )(tok2, comb, w2, b2)


def kernel(x, token_table, pos_table, seg_table, ln_w, ln_b):
    batch = x.shape[0]
    flat_idx = x.reshape(-1)
    tok = _sc_gather(token_table, flat_idx)
    tok2 = tok.reshape(batch * _HSEQ, _PK)
    seg_ids = (jnp.arange(_SEQ, dtype=jnp.int32) >= _HALF).astype(jnp.int32)
    comb1 = (pos_table + seg_table[seg_ids]).reshape(_HSEQ, _PK)
    comb = jnp.tile(comb1, (_PBATCH, 1))
    w2 = jnp.tile(ln_w.reshape(1, _EMB), (1, 2))
    b2 = jnp.tile(ln_b.reshape(1, _EMB), (1, 2))
    out2 = _tc_add_ln(tok2, comb, w2, b2)
    return out2.reshape(batch, _SEQ, _EMB)


# speedup vs baseline: 11.8489x; 1.3363x over previous
import functools

import jax
import jax.numpy as jnp
from jax import lax
from jax.experimental import pallas as pl
from jax.experimental.pallas import tpu as pltpu
from jax.experimental.pallas import tpu_sc as plsc

_MAX_LEN = 150
_SEQ = _MAX_LEN * 2 + 2
_EMB = 64
_HALF = _MAX_LEN + 1
_HSEQ = _SEQ // 2
_PK = 2 * _EMB

_GATHER_W = 128


def _sc_gather(token_table, flat_idx):
    n = flat_idx.shape[0]
    assert n % _GATHER_W == 0
    mesh = plsc.VectorSubcoreMesh(core_axis_name="c", subcore_axis_name="s")

    @functools.partial(
        pl.kernel,
        out_type=jax.ShapeDtypeStruct((n, _EMB), jnp.float32),
        mesh=mesh,
        compiler_params=pltpu.CompilerParams(use_tc_tiling_on_sc=False),
    )
    def gather_kernel(tab_hbm, idx_hbm, out_hbm):
        def body(i_vmem, o_vmem):
            pltpu.sync_copy(tab_hbm.at[i_vmem.at[0]], o_vmem)

        pltpu.emit_pipeline(
            body,
            grid=(n // _GATHER_W,),
            in_specs=[pl.BlockSpec((1, _GATHER_W), lambda i: (0, i))],
            out_specs=[pl.BlockSpec((_GATHER_W, _EMB), lambda i: (i, 0))],
            core_axis_name=("c", "s"),
            dimension_semantics=(pltpu.PARALLEL,),
        )(idx_hbm, out_hbm)

    return gather_kernel(token_table, flat_idx.reshape(1, n))


def _ln_t_body(tok_ref, comb_ref, w2_ref, b2_ref, out_ref):
    k_io = lax.broadcasted_iota(jnp.int32, (_PK, _PK), 0)
    l_io = lax.broadcasted_iota(jnp.int32, (_PK, _PK), 1)
    avg = jnp.where((k_io >= _EMB) == (l_io >= _EMB), 1.0 / _EMB, 0.0)
    avg = avg.astype(jnp.bfloat16)

    emb = tok_ref[...] + comb_ref[0]
    mean = jnp.dot(emb.astype(jnp.bfloat16), avg,
                   preferred_element_type=jnp.float32)
    sq = emb * emb
    meansq = jnp.dot(sq.astype(jnp.bfloat16), avg,
                     preferred_element_type=jnp.float32)
    var = meansq - mean * mean
    normed = (emb - mean) * lax.rsqrt(var + 1e-5)
    outv = normed * w2_ref[...] + b2_ref[...]
    t = outv.T
    out_ref[...] = t.reshape(2, _EMB, t.shape[1])


def _tc_ln_t(tok2, comb, w2, b2, batch):
    rows = tok2.shape[0]
    assert rows == _HSEQ * batch
    return pl.pallas_call(
        _ln_t_body,
        out_shape=jax.ShapeDtypeStruct((_SEQ, _EMB, batch), jnp.float32),
        grid=(_HSEQ,),
        in_specs=[
            pl.BlockSpec((batch, _PK), lambda i: (i, 0)),
            pl.BlockSpec((1, 1, _PK), lambda i: (i, 0, 0)),
            pl.BlockSpec((1, _PK), lambda i: (0, 0)),
            pl.BlockSpec((1, _PK), lambda i: (0, 0)),
        ],
        out_specs=pl.BlockSpec((2, _EMB, batch), lambda i: (i, 0, 0)),
    )(tok2, comb, w2, b2)


def kernel(x, token_table, pos_table, seg_table, ln_w, ln_b):
    batch = x.shape[0]
    idx_km = x.T.reshape(_HSEQ, 2, batch).transpose(0, 2, 1).reshape(-1)
    tok = _sc_gather(token_table, idx_km)
    tok2 = tok.reshape(_HSEQ * batch, _PK)
    seg_ids = (jnp.arange(_SEQ, dtype=jnp.int32) >= _HALF).astype(jnp.int32)
    comb = (pos_table + seg_table[seg_ids]).reshape(_HSEQ, 1, _PK)
    w2 = jnp.tile(ln_w.reshape(1, _EMB), (1, 2))
    b2 = jnp.tile(ln_b.reshape(1, _EMB), (1, 2))
    out3 = _tc_ln_t(tok2, comb, w2, b2, batch)
    return jnp.transpose(out3, (2, 0, 1))


# speedup vs baseline: 12.0892x; 1.0203x over previous
import functools

import jax
import jax.numpy as jnp
from jax import lax
from jax.experimental import pallas as pl
from jax.experimental.pallas import tpu as pltpu
from jax.experimental.pallas import tpu_sc as plsc

_MAX_LEN = 150
_SEQ = _MAX_LEN * 2 + 2
_EMB = 64
_HALF = _MAX_LEN + 1
_HSEQ = _SEQ // 2
_PK = 2 * _EMB

_GATHER_W = 128


def _sc_gather(token_table, flat_idx):
    n = flat_idx.shape[0]
    assert n % _GATHER_W == 0
    mesh = plsc.VectorSubcoreMesh(core_axis_name="c", subcore_axis_name="s")

    @functools.partial(
        pl.kernel,
        out_type=jax.ShapeDtypeStruct((n, _EMB), jnp.float32),
        mesh=mesh,
        compiler_params=pltpu.CompilerParams(use_tc_tiling_on_sc=False),
    )
    def gather_kernel(tab_hbm, idx_hbm, out_hbm):
        def body(i_vmem, o_vmem):
            pltpu.sync_copy(tab_hbm.at[i_vmem.at[0]], o_vmem)

        pltpu.emit_pipeline(
            body,
            grid=(n // _GATHER_W,),
            in_specs=[pl.BlockSpec((1, _GATHER_W), lambda i: (0, i))],
            out_specs=[pl.BlockSpec((_GATHER_W, _EMB), lambda i: (i, 0))],
            core_axis_name=("c", "s"),
            dimension_semantics=(pltpu.PARALLEL,),
        )(idx_hbm, out_hbm)

    return gather_kernel(token_table, flat_idx.reshape(1, n))


def _ln_t_body(tok_ref, comb_ref, w2_ref, b2_ref, out_ref):
    k_io = lax.broadcasted_iota(jnp.int32, (_PK, _PK), 0)
    l_io = lax.broadcasted_iota(jnp.int32, (_PK, _PK), 1)
    avg = jnp.where((k_io >= _EMB) == (l_io >= _EMB), 1.0 / _EMB, 0.0)
    avg = avg.astype(jnp.bfloat16)

    emb = tok_ref[...] + comb_ref[0]
    mean = jnp.dot(emb.astype(jnp.bfloat16), avg,
                   preferred_element_type=jnp.float32)
    sq = emb * emb
    meansq = jnp.dot(sq.astype(jnp.bfloat16), avg,
                     preferred_element_type=jnp.float32)
    var = meansq - mean * mean
    normed = (emb - mean) * lax.rsqrt(var + 1e-5)
    outv = normed * w2_ref[...] + b2_ref[...]
    t = outv.T
    out_ref[...] = t.reshape(2, _EMB, t.shape[1])


def _ln_t_body_aliased(tok_ref, comb_ref, w2_ref, b2_ref, prev_ref, out_ref):
    del prev_ref
    _ln_t_body(tok_ref, comb_ref, w2_ref, b2_ref, out_ref)


def _tc_ln_t_chunk(tok2_c, comb, w2, b2, batch, k0, k1, prev):
    nk = k1 - k0
    assert tok2_c.shape[0] == nk * batch
    out_shape = jax.ShapeDtypeStruct((_SEQ, _EMB, batch), jnp.float32)
    in_specs = [
        pl.BlockSpec((batch, _PK), lambda i: (i, 0)),
        pl.BlockSpec((1, 1, _PK), lambda i: (i + k0, 0, 0)),
        pl.BlockSpec((1, _PK), lambda i: (0, 0)),
        pl.BlockSpec((1, _PK), lambda i: (0, 0)),
    ]
    args = [tok2_c, comb, w2, b2]
    body = _ln_t_body
    alias = {}
    if prev is not None:
        in_specs.append(pl.BlockSpec(memory_space=pl.ANY))
        args.append(prev)
        body = _ln_t_body_aliased
        alias = {4: 0}
    return pl.pallas_call(
        body,
        out_shape=out_shape,
        grid=(nk,),
        in_specs=in_specs,
        out_specs=pl.BlockSpec((2, _EMB, batch), lambda i: (i + k0, 0, 0)),
        input_output_aliases=alias,
    )(*args)


def kernel(x, token_table, pos_table, seg_table, ln_w, ln_b):
    batch = x.shape[0]
    idx_km = x.T.reshape(_HSEQ, 2, batch).transpose(0, 2, 1).reshape(-1)
    seg_ids = (jnp.arange(_SEQ, dtype=jnp.int32) >= _HALF).astype(jnp.int32)
    comb = (pos_table + seg_table[seg_ids]).reshape(_HSEQ, 1, _PK)
    w2 = jnp.tile(ln_w.reshape(1, _EMB), (1, 2))
    b2 = jnp.tile(ln_b.reshape(1, _EMB), (1, 2))
    bounds = (0, 50, 100, _HSEQ)
    out3 = None
    for k0, k1 in zip(bounds[:-1], bounds[1:]):
        idx_c = idx_km[k0 * 2 * batch : k1 * 2 * batch]
        tok_c = _sc_gather(token_table, idx_c)
        tok2_c = tok_c.reshape((k1 - k0) * batch, _PK)
        out3 = _tc_ln_t_chunk(tok2_c, comb, w2, b2, batch, k0, k1, out3)
    return jnp.transpose(out3, (2, 0, 1))


# speedup vs baseline: 12.1450x; 1.0046x over previous
import functools

import jax
import jax.numpy as jnp
from jax import lax
from jax.experimental import pallas as pl
from jax.experimental.pallas import tpu as pltpu
from jax.experimental.pallas import tpu_sc as plsc

_MAX_LEN = 150
_SEQ = _MAX_LEN * 2 + 2
_EMB = 64
_HALF = _MAX_LEN + 1
_HSEQ = _SEQ // 2
_PK = 2 * _EMB

_GATHER_W = 128
_GATHERS_PER_STEP = 4


def _sc_gather(token_table, flat_idx):
    n = flat_idx.shape[0]
    assert n % _GATHER_W == 0
    mesh = plsc.VectorSubcoreMesh(core_axis_name="c", subcore_axis_name="s")

    @functools.partial(
        pl.kernel,
        out_type=jax.ShapeDtypeStruct((n, _EMB), jnp.float32),
        mesh=mesh,
        compiler_params=pltpu.CompilerParams(use_tc_tiling_on_sc=False),
    )
    def gather_kernel(tab_hbm, idx_hbm, out_hbm):
        def body(i_vmem, o_vmem):
            for t in range(_GATHERS_PER_STEP):
                sl = pl.ds(t * _GATHER_W, _GATHER_W)
                pltpu.sync_copy(tab_hbm.at[i_vmem.at[0, sl]], o_vmem.at[sl])

        step = _GATHER_W * _GATHERS_PER_STEP
        pltpu.emit_pipeline(
            body,
            grid=(n // step,),
            in_specs=[pl.BlockSpec((1, step), lambda i: (0, i))],
            out_specs=[pl.BlockSpec((step, _EMB), lambda i: (i, 0))],
            core_axis_name=("c", "s"),
            dimension_semantics=(pltpu.PARALLEL,),
        )(idx_hbm, out_hbm)

    return gather_kernel(token_table, flat_idx.reshape(1, n))


def _ln_t_body(tok_ref, comb_ref, w2_ref, b2_ref, out_ref):
    k_io = lax.broadcasted_iota(jnp.int32, (_PK, _PK), 0)
    l_io = lax.broadcasted_iota(jnp.int32, (_PK, _PK), 1)
    avg = jnp.where((k_io >= _EMB) == (l_io >= _EMB), 1.0 / _EMB, 0.0)
    avg = avg.astype(jnp.bfloat16)

    emb = tok_ref[...] + comb_ref[0]
    mean = jnp.dot(emb.astype(jnp.bfloat16), avg,
                   preferred_element_type=jnp.float32)
    sq = emb * emb
    meansq = jnp.dot(sq.astype(jnp.bfloat16), avg,
                     preferred_element_type=jnp.float32)
    var = meansq - mean * mean
    normed = (emb - mean) * lax.rsqrt(var + 1e-5)
    outv = normed * w2_ref[...] + b2_ref[...]
    t = outv.T
    out_ref[...] = t.reshape(2, _EMB, t.shape[1])


def _ln_t_body_aliased(tok_ref, comb_ref, w2_ref, b2_ref, prev_ref, out_ref):
    del prev_ref
    _ln_t_body(tok_ref, comb_ref, w2_ref, b2_ref, out_ref)


def _tc_ln_t_chunk(tok2_c, comb, w2, b2, batch, k0, k1, prev):
    nk = k1 - k0
    assert tok2_c.shape[0] == nk * batch
    out_shape = jax.ShapeDtypeStruct((_SEQ, _EMB, batch), jnp.float32)
    in_specs = [
        pl.BlockSpec((batch, _PK), lambda i: (i, 0)),
        pl.BlockSpec((1, 1, _PK), lambda i: (i + k0, 0, 0)),
        pl.BlockSpec((1, _PK), lambda i: (0, 0)),
        pl.BlockSpec((1, _PK), lambda i: (0, 0)),
    ]
    args = [tok2_c, comb, w2, b2]
    body = _ln_t_body
    alias = {}
    if prev is not None:
        in_specs.append(pl.BlockSpec(memory_space=pl.ANY))
        args.append(prev)
        body = _ln_t_body_aliased
        alias = {4: 0}
    return pl.pallas_call(
        body,
        out_shape=out_shape,
        grid=(nk,),
        in_specs=in_specs,
        out_specs=pl.BlockSpec((2, _EMB, batch), lambda i: (i + k0, 0, 0)),
        input_output_aliases=alias,
    )(*args)


def kernel(x, token_table, pos_table, seg_table, ln_w, ln_b):
    batch = x.shape[0]
    idx_km = x.T.reshape(_HSEQ, 2, batch).transpose(0, 2, 1).reshape(-1)
    seg_ids = (jnp.arange(_SEQ, dtype=jnp.int32) >= _HALF).astype(jnp.int32)
    comb = (pos_table + seg_table[seg_ids]).reshape(_HSEQ, 1, _PK)
    w2 = jnp.tile(ln_w.reshape(1, _EMB), (1, 2))
    b2 = jnp.tile(ln_b.reshape(1, _EMB), (1, 2))
    bounds = (0, 50, 100, _HSEQ)
    out3 = None
    for k0, k1 in zip(bounds[:-1], bounds[1:]):
        idx_c = idx_km[k0 * 2 * batch : k1 * 2 * batch]
        tok_c = _sc_gather(token_table, idx_c)
        tok2_c = tok_c.reshape((k1 - k0) * batch, _PK)
        out3 = _tc_ln_t_chunk(tok2_c, comb, w2, b2, batch, k0, k1, out3)
    return jnp.transpose(out3, (2, 0, 1))


# speedup vs baseline: 12.3999x; 1.0210x over previous
import functools

import jax
import jax.numpy as jnp
from jax import lax
from jax.experimental import pallas as pl
from jax.experimental.pallas import tpu as pltpu
from jax.experimental.pallas import tpu_sc as plsc

_MAX_LEN = 150
_SEQ = _MAX_LEN * 2 + 2
_EMB = 64
_HALF = _MAX_LEN + 1
_HSEQ = _SEQ // 2
_PK = 2 * _EMB

_GATHER_W = 128
_GATHERS_PER_STEP = 4


def _sc_gather(token_table, flat_idx):
    n = flat_idx.shape[0]
    assert n % _GATHER_W == 0
    mesh = plsc.VectorSubcoreMesh(core_axis_name="c", subcore_axis_name="s")

    @functools.partial(
        pl.kernel,
        out_type=jax.ShapeDtypeStruct((n, _EMB), jnp.float32),
        mesh=mesh,
        compiler_params=pltpu.CompilerParams(use_tc_tiling_on_sc=False),
    )
    def gather_kernel(tab_hbm, idx_hbm, out_hbm):
        def body(i_vmem, o_vmem):
            for t in range(_GATHERS_PER_STEP):
                sl = pl.ds(t * _GATHER_W, _GATHER_W)
                pltpu.sync_copy(tab_hbm.at[i_vmem.at[0, sl]], o_vmem.at[sl])

        step = _GATHER_W * _GATHERS_PER_STEP
        pltpu.emit_pipeline(
            body,
            grid=(n // step,),
            in_specs=[pl.BlockSpec((1, step), lambda i: (0, i))],
            out_specs=[pl.BlockSpec((step, _EMB), lambda i: (i, 0))],
            core_axis_name=("c", "s"),
            dimension_semantics=(pltpu.PARALLEL,),
        )(idx_hbm, out_hbm)

    return gather_kernel(token_table, flat_idx.reshape(1, n))


def _ln_t_body(tok_ref, comb_ref, w2_ref, b2_ref, out_ref):
    k_io = lax.broadcasted_iota(jnp.int32, (_PK, _PK), 0)
    l_io = lax.broadcasted_iota(jnp.int32, (_PK, _PK), 1)
    avg = jnp.where((k_io >= _EMB) == (l_io >= _EMB), 1.0 / _EMB, 0.0)
    avg = avg.astype(jnp.bfloat16)

    emb = tok_ref[...] + comb_ref[0]
    mean = jnp.dot(emb.astype(jnp.bfloat16), avg,
                   preferred_element_type=jnp.float32)
    sq = emb * emb
    meansq = jnp.dot(sq.astype(jnp.bfloat16), avg,
                     preferred_element_type=jnp.float32)
    var = meansq - mean * mean
    normed = (emb - mean) * lax.rsqrt(var + 1e-5)
    outv = normed * w2_ref[...] + b2_ref[...]
    t = outv.T
    out_ref[...] = t.reshape(2, _EMB, t.shape[1])


def _ln_t_body_aliased(tok_ref, comb_ref, w2_ref, b2_ref, prev_ref, out_ref):
    del prev_ref
    _ln_t_body(tok_ref, comb_ref, w2_ref, b2_ref, out_ref)


def _tc_ln_t_chunk(tok2_c, comb, w2, b2, batch, k0, k1, prev):
    nk = k1 - k0
    assert tok2_c.shape[0] == nk * batch
    out_shape = jax.ShapeDtypeStruct((_SEQ, _EMB, batch), jnp.float32)
    in_specs = [
        pl.BlockSpec((batch, _PK), lambda i: (i, 0)),
        pl.BlockSpec((1, 1, _PK), lambda i: (i + k0, 0, 0)),
        pl.BlockSpec((1, _PK), lambda i: (0, 0)),
        pl.BlockSpec((1, _PK), lambda i: (0, 0)),
    ]
    args = [tok2_c, comb, w2, b2]
    body = _ln_t_body
    alias = {}
    if prev is not None:
        in_specs.append(pl.BlockSpec(memory_space=pl.ANY))
        args.append(prev)
        body = _ln_t_body_aliased
        alias = {4: 0}
    return pl.pallas_call(
        body,
        out_shape=out_shape,
        grid=(nk,),
        in_specs=in_specs,
        out_specs=pl.BlockSpec((2, _EMB, batch), lambda i: (i + k0, 0, 0)),
        input_output_aliases=alias,
    )(*args)


def kernel(x, token_table, pos_table, seg_table, ln_w, ln_b):
    batch = x.shape[0]
    idx_km = x.T.reshape(_HSEQ, 2, batch).transpose(0, 2, 1).reshape(-1)
    seg_ids = (jnp.arange(_SEQ, dtype=jnp.int32) >= _HALF).astype(jnp.int32)
    comb = (pos_table + seg_table[seg_ids]).reshape(_HSEQ, 1, _PK)
    w2 = jnp.tile(ln_w.reshape(1, _EMB), (1, 2))
    b2 = jnp.tile(ln_b.reshape(1, _EMB), (1, 2))
    bounds = (0, 30, 60, 90, 120, _HSEQ)
    out3 = None
    for k0, k1 in zip(bounds[:-1], bounds[1:]):
        idx_c = idx_km[k0 * 2 * batch : k1 * 2 * batch]
        tok_c = _sc_gather(token_table, idx_c)
        tok2_c = tok_c.reshape((k1 - k0) * batch, _PK)
        out3 = _tc_ln_t_chunk(tok2_c, comb, w2, b2, batch, k0, k1, out3)
    return jnp.transpose(out3, (2, 0, 1))


# speedup vs baseline: 12.4427x; 1.0035x over previous
import functools

import jax
import jax.numpy as jnp
from jax import lax
from jax.experimental import pallas as pl
from jax.experimental.pallas import tpu as pltpu
from jax.experimental.pallas import tpu_sc as plsc

_MAX_LEN = 150
_SEQ = _MAX_LEN * 2 + 2
_EMB = 64
_HALF = _MAX_LEN + 1
_HSEQ = _SEQ // 2
_PK = 2 * _EMB

_GATHER_W = 128
_GATHERS_PER_STEP = 4


def _sc_gather(token_table, flat_idx):
    n = flat_idx.shape[0]
    assert n % _GATHER_W == 0
    mesh = plsc.VectorSubcoreMesh(core_axis_name="c", subcore_axis_name="s")

    @functools.partial(
        pl.kernel,
        out_type=jax.ShapeDtypeStruct((n, _EMB), jnp.float32),
        mesh=mesh,
        compiler_params=pltpu.CompilerParams(use_tc_tiling_on_sc=False),
    )
    def gather_kernel(tab_hbm, idx_hbm, out_hbm):
        def body(i_vmem, o_vmem):
            for t in range(_GATHERS_PER_STEP):
                sl = pl.ds(t * _GATHER_W, _GATHER_W)
                pltpu.sync_copy(tab_hbm.at[i_vmem.at[0, sl]], o_vmem.at[sl])

        step = _GATHER_W * _GATHERS_PER_STEP
        pltpu.emit_pipeline(
            body,
            grid=(n // step,),
            in_specs=[pl.BlockSpec((1, step), lambda i: (0, i))],
            out_specs=[pl.BlockSpec((step, _EMB), lambda i: (i, 0))],
            core_axis_name=("c", "s"),
            dimension_semantics=(pltpu.PARALLEL,),
        )(idx_hbm, out_hbm)

    return gather_kernel(token_table, flat_idx.reshape(1, n))


def _ln_t_body(tok_ref, comb_ref, w2_ref, b2_ref, out_ref):
    k_io = lax.broadcasted_iota(jnp.int32, (_PK, _PK), 0)
    l_io = lax.broadcasted_iota(jnp.int32, (_PK, _PK), 1)
    avg = jnp.where((k_io >= _EMB) == (l_io >= _EMB), 1.0 / _EMB, 0.0)
    avg = avg.astype(jnp.bfloat16)

    emb = tok_ref[...] + comb_ref[0]
    mean = jnp.dot(emb.astype(jnp.bfloat16), avg,
                   preferred_element_type=jnp.float32)
    sq = emb * emb
    meansq = jnp.dot(sq.astype(jnp.bfloat16), avg,
                     preferred_element_type=jnp.float32)
    var = meansq - mean * mean
    normed = (emb - mean) * lax.rsqrt(var + 1e-5)
    outv = normed * w2_ref[...] + b2_ref[...]
    t = outv.T
    out_ref[...] = t.reshape(2, _EMB, t.shape[1])


def _ln_t_body_aliased(tok_ref, comb_ref, w2_ref, b2_ref, prev_ref, out_ref):
    del prev_ref
    _ln_t_body(tok_ref, comb_ref, w2_ref, b2_ref, out_ref)


def _tc_ln_t_chunk(tok2_c, comb, w2, b2, batch, k0, k1, prev):
    nk = k1 - k0
    assert tok2_c.shape[0] == nk * batch
    out_shape = jax.ShapeDtypeStruct((_SEQ, _EMB, batch), jnp.float32)
    in_specs = [
        pl.BlockSpec((batch, _PK), lambda i: (i, 0)),
        pl.BlockSpec((1, 1, _PK), lambda i: (i + k0, 0, 0)),
        pl.BlockSpec((1, _PK), lambda i: (0, 0)),
        pl.BlockSpec((1, _PK), lambda i: (0, 0)),
    ]
    args = [tok2_c, comb, w2, b2]
    body = _ln_t_body
    alias = {}
    if prev is not None:
        in_specs.append(pl.BlockSpec(memory_space=pl.ANY))
        args.append(prev)
        body = _ln_t_body_aliased
        alias = {4: 0}
    return pl.pallas_call(
        body,
        out_shape=out_shape,
        grid=(nk,),
        in_specs=in_specs,
        out_specs=pl.BlockSpec((2, _EMB, batch), lambda i: (i + k0, 0, 0)),
        input_output_aliases=alias,
    )(*args)


def kernel(x, token_table, pos_table, seg_table, ln_w, ln_b):
    batch = x.shape[0]
    idx_km = x.T.reshape(_HSEQ, 2, batch).transpose(0, 2, 1).reshape(-1)
    seg_ids = (jnp.arange(_SEQ, dtype=jnp.int32) >= _HALF).astype(jnp.int32)
    comb = (pos_table + seg_table[seg_ids]).reshape(_HSEQ, 1, _PK)
    w2 = jnp.tile(ln_w.reshape(1, _EMB), (1, 2))
    b2 = jnp.tile(ln_b.reshape(1, _EMB), (1, 2))
    bounds = (0, 19, 38, 57, 76, 95, 114, 133, _HSEQ)
    out3 = None
    for k0, k1 in zip(bounds[:-1], bounds[1:]):
        idx_c = idx_km[k0 * 2 * batch : k1 * 2 * batch]
        tok_c = _sc_gather(token_table, idx_c)
        tok2_c = tok_c.reshape((k1 - k0) * batch, _PK)
        out3 = _tc_ln_t_chunk(tok2_c, comb, w2, b2, batch, k0, k1, out3)
    return jnp.transpose(out3, (2, 0, 1))


# speedup vs baseline: 12.4796x; 1.0030x over previous
import functools

import jax
import jax.numpy as jnp
from jax import lax
from jax.experimental import pallas as pl
from jax.experimental.pallas import tpu as pltpu
from jax.experimental.pallas import tpu_sc as plsc

_MAX_LEN = 150
_SEQ = _MAX_LEN * 2 + 2
_EMB = 64
_HALF = _MAX_LEN + 1
_HSEQ = _SEQ // 2
_PK = 2 * _EMB

_GATHER_W = 128
_GATHERS_PER_STEP = 4


def _sc_gather(token_table, flat_idx):
    n = flat_idx.shape[0]
    assert n % _GATHER_W == 0
    mesh = plsc.VectorSubcoreMesh(core_axis_name="c", subcore_axis_name="s")

    @functools.partial(
        pl.kernel,
        out_type=jax.ShapeDtypeStruct((n, _EMB), jnp.float32),
        mesh=mesh,
        compiler_params=pltpu.CompilerParams(use_tc_tiling_on_sc=False),
    )
    def gather_kernel(tab_hbm, idx_hbm, out_hbm):
        def body(i_vmem, o_vmem):
            for t in range(_GATHERS_PER_STEP):
                sl = pl.ds(t * _GATHER_W, _GATHER_W)
                pltpu.sync_copy(tab_hbm.at[i_vmem.at[0, sl]], o_vmem.at[sl])

        step = _GATHER_W * _GATHERS_PER_STEP
        pltpu.emit_pipeline(
            body,
            grid=(n // step,),
            in_specs=[pl.BlockSpec((1, step), lambda i: (0, i))],
            out_specs=[pl.BlockSpec((step, _EMB), lambda i: (i, 0))],
            core_axis_name=("c", "s"),
            dimension_semantics=(pltpu.PARALLEL,),
        )(idx_hbm, out_hbm)

    return gather_kernel(token_table, flat_idx.reshape(1, n))


def _ln_t_body(tok_ref, comb_ref, w2_ref, b2_ref, out_ref):
    k_io = lax.broadcasted_iota(jnp.int32, (_PK, _PK), 0)
    l_io = lax.broadcasted_iota(jnp.int32, (_PK, _PK), 1)
    avg = jnp.where((k_io >= _EMB) == (l_io >= _EMB), 1.0 / _EMB, 0.0)
    avg = avg.astype(jnp.bfloat16)

    emb = tok_ref[...] + comb_ref[0]
    mean = jnp.dot(emb.astype(jnp.bfloat16), avg,
                   preferred_element_type=jnp.float32)
    sq = emb * emb
    meansq = jnp.dot(sq.astype(jnp.bfloat16), avg,
                     preferred_element_type=jnp.float32)
    var = meansq - mean * mean
    normed = (emb - mean) * lax.rsqrt(var + 1e-5)
    outv = normed * w2_ref[...] + b2_ref[...]
    t = outv.T
    out_ref[...] = t.reshape(2, _EMB, t.shape[1])


def _ln_t_body_aliased(tok_ref, comb_ref, w2_ref, b2_ref, prev_ref, out_ref):
    del prev_ref
    _ln_t_body(tok_ref, comb_ref, w2_ref, b2_ref, out_ref)


def _tc_ln_t_chunk(tok2_c, comb, w2, b2, batch, k0, k1, prev):
    nk = k1 - k0
    assert tok2_c.shape[0] == nk * batch
    out_shape = jax.ShapeDtypeStruct((_SEQ, _EMB, batch), jnp.float32)
    in_specs = [
        pl.BlockSpec((batch, _PK), lambda i: (i, 0)),
        pl.BlockSpec((1, 1, _PK), lambda i: (i + k0, 0, 0)),
        pl.BlockSpec((1, _PK), lambda i: (0, 0)),
        pl.BlockSpec((1, _PK), lambda i: (0, 0)),
    ]
    args = [tok2_c, comb, w2, b2]
    body = _ln_t_body
    alias = {}
    if prev is not None:
        in_specs.append(pl.BlockSpec(memory_space=pl.ANY))
        args.append(prev)
        body = _ln_t_body_aliased
        alias = {4: 0}
    return pl.pallas_call(
        body,
        out_shape=out_shape,
        grid=(nk,),
        in_specs=in_specs,
        out_specs=pl.BlockSpec((2, _EMB, batch), lambda i: (i + k0, 0, 0)),
        input_output_aliases=alias,
    )(*args)


def kernel(x, token_table, pos_table, seg_table, ln_w, ln_b):
    batch = x.shape[0]
    idx_km = x.T.reshape(_HSEQ, 2, batch).transpose(0, 2, 1).reshape(-1)
    seg_ids = (jnp.arange(_SEQ, dtype=jnp.int32) >= _HALF).astype(jnp.int32)
    comb = (pos_table + seg_table[seg_ids]).reshape(_HSEQ, 1, _PK)
    w2 = jnp.tile(ln_w.reshape(1, _EMB), (1, 2))
    b2 = jnp.tile(ln_b.reshape(1, _EMB), (1, 2))
    bounds = (0, 14, 28, 42, 56, 70, 84, 98, 112, 126, 140, _HSEQ)
    out3 = None
    for k0, k1 in zip(bounds[:-1], bounds[1:]):
        idx_c = idx_km[k0 * 2 * batch : k1 * 2 * batch]
        tok_c = _sc_gather(token_table, idx_c)
        tok2_c = tok_c.reshape((k1 - k0) * batch, _PK)
        out3 = _tc_ln_t_chunk(tok2_c, comb, w2, b2, batch, k0, k1, out3)
    return jnp.transpose(out3, (2, 0, 1))
